# baseline (speedup 1.0000x reference)
import jax
import jax.numpy as jnp
from jax.experimental import pallas as pl
from jax.experimental.pallas import tpu as pltpu

_DIM = 256
_K = 1024
_TOK_BLK = 1024
_K_CHUNK = 32


def _vq_body(x_ref, cb_ref, cb2_ref, cbb_ref, q_ref, idx_ref, comm_ref,
             perp_ref, counts_scr, csum_scr):
    b = pl.program_id(0)
    tc = pl.program_id(1)
    nb = pl.num_programs(0)
    ntc = pl.num_programs(1)
    first = jnp.logical_and(b == 0, tc == 0)
    last = jnp.logical_and(b == nb - 1, tc == ntc - 1)

    xT = x_ref[0]

    mm2T = jnp.dot(cb2_ref[...], xT, preferred_element_type=jnp.float32)
    xsq = jnp.sum(xT * xT, axis=0)

    m = None
    idx = None
    for kc in range(_K // _K_CHUNK):
        sl = pl.ds(kc * _K_CHUNK, _K_CHUNK)
        cb_c = cb_ref[sl, :]
        cbn_c = jnp.sum(cb_c * cb_c, axis=1)
        d_c = (xsq[None, :] - mm2T[kc * _K_CHUNK:(kc + 1) * _K_CHUNK, :]) \
            + cbn_c[:, None]
        cmin = jnp.min(d_c, axis=0)
        iota_c = jax.lax.broadcasted_iota(jnp.int32, (_K_CHUNK, _TOK_BLK), 0) \
            + kc * _K_CHUNK
        idxc = jnp.min(jnp.where(d_c == cmin[None, :], iota_c, _K), axis=0)
        if kc == 0:
            m, idx = cmin, idxc
        else:
            idx = jnp.where(cmin < m, idxc, idx)
            m = jnp.minimum(m, cmin)
    idx_ref[0, 0, :] = idx

    iota = jax.lax.broadcasted_iota(jnp.int32, (_K, _TOK_BLK), 0)

    hit = iota == idx[None, :]
    onehot = hit.astype(jnp.float32)
    qT = jax.lax.dot_general(cbb_ref[...], hit.astype(jnp.bfloat16),
                             (((0,), (0,)), ((), ())),
                             preferred_element_type=jnp.float32)
    q_ref[0] = qT

    blk_csum = jnp.sum(m)

    @pl.when(first)
    def _():
        counts_scr[...] = onehot
        csum_scr[0] = blk_csum

    @pl.when(jnp.logical_not(first))
    def _():
        counts_scr[...] = counts_scr[...] + onehot
        csum_scr[0] = csum_scr[0] + blk_csum

    @pl.when(last)
    def _():
        counts = jnp.sum(counts_scr[...], axis=1)
        total = jnp.sum(counts)
        probs = counts / jnp.maximum(total, 1.0)
        ent = -jnp.sum(probs * jnp.log(probs + 1e-10))
        perp_ref[...] = jnp.full((1, 1), jnp.exp(ent), jnp.float32)
        n_elems = nb * ntc * _TOK_BLK * _DIM
        comm_ref[...] = jnp.full((1, 1), csum_scr[0] / n_elems, jnp.float32)

    @pl.when(jnp.logical_not(last))
    def _():
        perp_ref[...] = jnp.zeros((1, 1), jnp.float32)
        comm_ref[...] = jnp.zeros((1, 1), jnp.float32)


def kernel(x, codebook):
    b, d, t = x.shape
    n_tc = t // _TOK_BLK
    grid = (b, n_tc)

    q, idx3, comm, perp = pl.pallas_call(
        _vq_body,
        grid=grid,
        in_specs=[
            pl.BlockSpec((1, d, _TOK_BLK), lambda i, j: (i, 0, j)),
            pl.BlockSpec((_K, d), lambda i, j: (0, 0)),
            pl.BlockSpec((_K, d), lambda i, j: (0, 0)),
            pl.BlockSpec((_K, d), lambda i, j: (0, 0)),
        ],
        out_specs=[
            pl.BlockSpec((1, d, _TOK_BLK), lambda i, j: (i, 0, j)),
            pl.BlockSpec((1, 1, _TOK_BLK), lambda i, j: (i, 0, j)),
            pl.BlockSpec((1, 1), lambda i, j: (0, 0)),
            pl.BlockSpec((1, 1), lambda i, j: (0, 0)),
        ],
        out_shape=[
            jax.ShapeDtypeStruct((b, d, t), jnp.float32),
            jax.ShapeDtypeStruct((b, 1, t), jnp.int32),
            jax.ShapeDtypeStruct((1, 1), jnp.float32),
            jax.ShapeDtypeStruct((1, 1), jnp.float32),
        ],
        scratch_shapes=[
            pltpu.VMEM((_K, _TOK_BLK), jnp.float32),
            pltpu.SMEM((1,), jnp.float32),
        ],
        compiler_params=pltpu.CompilerParams(
            dimension_semantics=("arbitrary", "arbitrary"),
        ),
    )(x, codebook, codebook * 2.0, codebook.astype(jnp.bfloat16))

    indices_2d = idx3.reshape(b, t)
    codebook_loss = jnp.zeros((), dtype=jnp.float32)
    return (q, indices_2d, codebook_loss, comm.reshape(()), perp.reshape(()))


# speedup vs baseline: 1.0159x; 1.0159x over previous
import jax
import jax.numpy as jnp
from jax.experimental import pallas as pl
from jax.experimental.pallas import tpu as pltpu

_DIM = 256
_K = 1024
_TOK_BLK = 1024
_K_CHUNK = 64


def _vq_body(x_ref, cb_ref, cb2_ref, cbb_ref, q_ref, idx_ref, comm_ref,
             perp_ref, counts_scr, csum_scr):
    b = pl.program_id(0)
    tc = pl.program_id(1)
    nb = pl.num_programs(0)
    ntc = pl.num_programs(1)
    first = jnp.logical_and(b == 0, tc == 0)
    last = jnp.logical_and(b == nb - 1, tc == ntc - 1)

    xT = x_ref[0]

    mm2T = jnp.dot(cb2_ref[...], xT, preferred_element_type=jnp.float32)
    xsq = jnp.sum(xT * xT, axis=0)

    m = None
    idx = None
    for kc in range(_K // _K_CHUNK):
        sl = pl.ds(kc * _K_CHUNK, _K_CHUNK)
        cb_c = cb_ref[sl, :]
        cbn_c = jnp.sum(cb_c * cb_c, axis=1)
        d_c = (xsq[None, :] - mm2T[kc * _K_CHUNK:(kc + 1) * _K_CHUNK, :]) \
            + cbn_c[:, None]
        cmin = jnp.min(d_c, axis=0)
        iota_c = jax.lax.broadcasted_iota(jnp.int32, (_K_CHUNK, _TOK_BLK), 0) \
            + kc * _K_CHUNK
        idxc = jnp.min(jnp.where(d_c == cmin[None, :], iota_c, _K), axis=0)
        if kc == 0:
            m, idx = cmin, idxc
        else:
            idx = jnp.where(cmin < m, idxc, idx)
            m = jnp.minimum(m, cmin)
    idx_ref[0, 0, :] = idx

    iota = jax.lax.broadcasted_iota(jnp.int32, (_K, _TOK_BLK), 0)

    hit_b = (iota == idx[None, :]).astype(jnp.bfloat16)
    qT = jax.lax.dot_general(cbb_ref[...], hit_b,
                             (((0,), (0,)), ((), ())),
                             preferred_element_type=jnp.float32)
    q_ref[0] = qT

    ones_b = jnp.ones((_TOK_BLK, 128), jnp.bfloat16)
    cnt_part = jax.lax.dot_general(hit_b, ones_b, (((1,), (0,)), ((), ())),
                                   preferred_element_type=jnp.float32)

    blk_csum = jnp.sum(m)

    @pl.when(first)
    def _():
        counts_scr[...] = cnt_part
        csum_scr[0] = blk_csum

    @pl.when(jnp.logical_not(first))
    def _():
        counts_scr[...] = counts_scr[...] + cnt_part
        csum_scr[0] = csum_scr[0] + blk_csum

    @pl.when(last)
    def _():
        counts = counts_scr[:, 0]
        total = jnp.sum(counts)
        probs = counts / jnp.maximum(total, 1.0)
        ent = -jnp.sum(probs * jnp.log(probs + 1e-10))
        perp_ref[...] = jnp.full((1, 1), jnp.exp(ent), jnp.float32)
        n_elems = nb * ntc * _TOK_BLK * _DIM
        comm_ref[...] = jnp.full((1, 1), csum_scr[0] / n_elems, jnp.float32)

    @pl.when(jnp.logical_not(last))
    def _():
        perp_ref[...] = jnp.zeros((1, 1), jnp.float32)
        comm_ref[...] = jnp.zeros((1, 1), jnp.float32)


def kernel(x, codebook):
    b, d, t = x.shape
    n_tc = t // _TOK_BLK
    grid = (b, n_tc)

    q, idx3, comm, perp = pl.pallas_call(
        _vq_body,
        grid=grid,
        in_specs=[
            pl.BlockSpec((1, d, _TOK_BLK), lambda i, j: (i, 0, j)),
            pl.BlockSpec((_K, d), lambda i, j: (0, 0)),
            pl.BlockSpec((_K, d), lambda i, j: (0, 0)),
            pl.BlockSpec((_K, d), lambda i, j: (0, 0)),
        ],
        out_specs=[
            pl.BlockSpec((1, d, _TOK_BLK), lambda i, j: (i, 0, j)),
            pl.BlockSpec((1, 1, _TOK_BLK), lambda i, j: (i, 0, j)),
            pl.BlockSpec((1, 1), lambda i, j: (0, 0)),
            pl.BlockSpec((1, 1), lambda i, j: (0, 0)),
        ],
        out_shape=[
            jax.ShapeDtypeStruct((b, d, t), jnp.float32),
            jax.ShapeDtypeStruct((b, 1, t), jnp.int32),
            jax.ShapeDtypeStruct((1, 1), jnp.float32),
            jax.ShapeDtypeStruct((1, 1), jnp.float32),
        ],
        scratch_shapes=[
            pltpu.VMEM((_K, 128), jnp.float32),
            pltpu.SMEM((1,), jnp.float32),
        ],
        compiler_params=pltpu.CompilerParams(
            dimension_semantics=("arbitrary", "arbitrary"),
        ),
    )(x, codebook, codebook * 2.0, codebook.astype(jnp.bfloat16))

    indices_2d = idx3.reshape(b, t)
    codebook_loss = jnp.zeros((), dtype=jnp.float32)
    return (q, indices_2d, codebook_loss, comm.reshape(()), perp.reshape(()))


# speedup vs baseline: 1.1338x; 1.1161x over previous
import jax
import jax.numpy as jnp
from jax.experimental import pallas as pl
from jax.experimental.pallas import tpu as pltpu

_DIM = 256
_K = 1024
_TOK_BLK = 1024
_K_CHUNK = 64


def _vq_body(x_ref, cb_ref, cb2_ref, cbb_ref, q_ref, idx_ref, comm_ref,
             perp_ref, counts_scr, csum_scr):
    b = pl.program_id(0)
    tc = pl.program_id(1)
    nb = pl.num_programs(0)
    ntc = pl.num_programs(1)
    first = jnp.logical_and(b == 0, tc == 0)
    last = jnp.logical_and(b == nb - 1, tc == ntc - 1)

    xT = x_ref[0]

    mm2T = jnp.dot(cb2_ref[...], xT, preferred_element_type=jnp.float32)
    xsq = jnp.sum(xT * xT, axis=0)

    m = None
    idx = None
    for kc in range(_K // _K_CHUNK):
        sl = pl.ds(kc * _K_CHUNK, _K_CHUNK)
        cb_c = cb_ref[sl, :]
        cbn_c = jnp.sum(cb_c * cb_c, axis=1)
        d_c = (xsq[None, :] - mm2T[kc * _K_CHUNK:(kc + 1) * _K_CHUNK, :]) \
            + cbn_c[:, None]
        cmin = jnp.min(d_c, axis=0)
        iota_c = jax.lax.broadcasted_iota(jnp.int32, (_K_CHUNK, _TOK_BLK), 0) \
            + kc * _K_CHUNK
        idxc = jnp.min(jnp.where(d_c == cmin[None, :], iota_c, _K), axis=0)
        if kc == 0:
            m, idx = cmin, idxc
        else:
            idx = jnp.where(cmin < m, idxc, idx)
            m = jnp.minimum(m, cmin)
    idx_ref[0, 0, :] = idx

    iota = jax.lax.broadcasted_iota(jnp.int32, (_K, _TOK_BLK), 0)

    hit = iota == idx[None, :]
    qT = jax.lax.dot_general(cbb_ref[...], hit.astype(jnp.bfloat16),
                             (((0,), (0,)), ((), ())),
                             preferred_element_type=jnp.float32)
    q_ref[0] = qT

    oh_i = hit.astype(jnp.int16)
    blk_csum = jnp.sum(m)

    @pl.when(first)
    def _():
        counts_scr[...] = oh_i
        csum_scr[0] = blk_csum

    @pl.when(jnp.logical_not(first))
    def _():
        counts_scr[...] = counts_scr[...] + oh_i
        csum_scr[0] = csum_scr[0] + blk_csum

    @pl.when(last)
    def _():
        counts = jnp.sum(counts_scr[...].astype(jnp.float32), axis=1)
        total = jnp.sum(counts)
        probs = counts / jnp.maximum(total, 1.0)
        ent = -jnp.sum(probs * jnp.log(probs + 1e-10))
        perp_ref[...] = jnp.full((1, 1), jnp.exp(ent), jnp.float32)
        n_elems = nb * ntc * _TOK_BLK * _DIM
        comm_ref[...] = jnp.full((1, 1), csum_scr[0] / n_elems, jnp.float32)

    @pl.when(jnp.logical_not(last))
    def _():
        perp_ref[...] = jnp.zeros((1, 1), jnp.float32)
        comm_ref[...] = jnp.zeros((1, 1), jnp.float32)


def kernel(x, codebook):
    b, d, t = x.shape
    n_tc = t // _TOK_BLK
    grid = (b, n_tc)

    q, idx3, comm, perp = pl.pallas_call(
        _vq_body,
        grid=grid,
        in_specs=[
            pl.BlockSpec((1, d, _TOK_BLK), lambda i, j: (i, 0, j)),
            pl.BlockSpec((_K, d), lambda i, j: (0, 0)),
            pl.BlockSpec((_K, d), lambda i, j: (0, 0)),
            pl.BlockSpec((_K, d), lambda i, j: (0, 0)),
        ],
        out_specs=[
            pl.BlockSpec((1, d, _TOK_BLK), lambda i, j: (i, 0, j)),
            pl.BlockSpec((1, 1, _TOK_BLK), lambda i, j: (i, 0, j)),
            pl.BlockSpec((1, 1), lambda i, j: (0, 0)),
            pl.BlockSpec((1, 1), lambda i, j: (0, 0)),
        ],
        out_shape=[
            jax.ShapeDtypeStruct((b, d, t), jnp.float32),
            jax.ShapeDtypeStruct((b, 1, t), jnp.int32),
            jax.ShapeDtypeStruct((1, 1), jnp.float32),
            jax.ShapeDtypeStruct((1, 1), jnp.float32),
        ],
        scratch_shapes=[
            pltpu.VMEM((_K, _TOK_BLK), jnp.int16),
            pltpu.SMEM((1,), jnp.float32),
        ],
        compiler_params=pltpu.CompilerParams(
            dimension_semantics=("arbitrary", "arbitrary"),
        ),
    )(x, codebook, codebook * 2.0, codebook.astype(jnp.bfloat16))

    indices_2d = idx3.reshape(b, t)
    codebook_loss = jnp.zeros((), dtype=jnp.float32)
    return (q, indices_2d, codebook_loss, comm.reshape(()), perp.reshape(()))


# speedup vs baseline: 1.1835x; 1.0438x over previous
import jax
import jax.numpy as jnp
from jax.experimental import pallas as pl
from jax.experimental.pallas import tpu as pltpu

_DIM = 256
_K = 1024
_TOK_BLK = 1024
_K_CHUNK = 64
_B_BLK = 2


def _vq_body(x_ref, cb_ref, cb2_ref, cbb_ref, q_ref, idx_ref, comm_ref,
             perp_ref, counts_scr, csum_scr):
    b = pl.program_id(0)
    tc = pl.program_id(1)
    nb = pl.num_programs(0)
    ntc = pl.num_programs(1)
    first = jnp.logical_and(b == 0, tc == 0)
    last = jnp.logical_and(b == nb - 1, tc == ntc - 1)

    oh_i = None
    blk_csum = None
    for b2 in range(_B_BLK):
        xT = x_ref[b2]

        mm2T = jnp.dot(cb2_ref[...], xT, preferred_element_type=jnp.float32)
        xsq = jnp.sum(xT * xT, axis=0)

        m = None
        idx = None
        for kc in range(_K // _K_CHUNK):
            sl = pl.ds(kc * _K_CHUNK, _K_CHUNK)
            cb_c = cb_ref[sl, :]
            cbn_c = jnp.sum(cb_c * cb_c, axis=1)
            d_c = (xsq[None, :] - mm2T[kc * _K_CHUNK:(kc + 1) * _K_CHUNK, :]) \
                + cbn_c[:, None]
            cmin = jnp.min(d_c, axis=0)
            iota_c = jax.lax.broadcasted_iota(
                jnp.int32, (_K_CHUNK, _TOK_BLK), 0) + kc * _K_CHUNK
            idxc = jnp.min(jnp.where(d_c == cmin[None, :], iota_c, _K), axis=0)
            if kc == 0:
                m, idx = cmin, idxc
            else:
                idx = jnp.where(cmin < m, idxc, idx)
                m = jnp.minimum(m, cmin)
        idx_ref[b2, 0, :] = idx

        iota = jax.lax.broadcasted_iota(jnp.int32, (_K, _TOK_BLK), 0)
        hit = iota == idx[None, :]
        qT = jax.lax.dot_general(cbb_ref[...], hit.astype(jnp.bfloat16),
                                 (((0,), (0,)), ((), ())),
                                 preferred_element_type=jnp.float32)
        q_ref[b2] = qT

        if b2 == 0:
            oh_i = hit.astype(jnp.int16)
            blk_csum = jnp.sum(m)
        else:
            oh_i = oh_i + hit.astype(jnp.int16)
            blk_csum = blk_csum + jnp.sum(m)

    @pl.when(first)
    def _():
        counts_scr[...] = oh_i
        csum_scr[0] = blk_csum

    @pl.when(jnp.logical_not(first))
    def _():
        counts_scr[...] = counts_scr[...] + oh_i
        csum_scr[0] = csum_scr[0] + blk_csum

    @pl.when(last)
    def _():
        counts = jnp.sum(counts_scr[...].astype(jnp.float32), axis=1)
        total = jnp.sum(counts)
        probs = counts / jnp.maximum(total, 1.0)
        ent = -jnp.sum(probs * jnp.log(probs + 1e-10))
        perp_ref[...] = jnp.full((1, 1), jnp.exp(ent), jnp.float32)
        n_elems = nb * ntc * _B_BLK * _TOK_BLK * _DIM
        comm_ref[...] = jnp.full((1, 1), csum_scr[0] / n_elems, jnp.float32)

    @pl.when(jnp.logical_not(last))
    def _():
        perp_ref[...] = jnp.zeros((1, 1), jnp.float32)
        comm_ref[...] = jnp.zeros((1, 1), jnp.float32)


def kernel(x, codebook):
    b, d, t = x.shape
    n_tc = t // _TOK_BLK
    grid = (b // _B_BLK, n_tc)

    q, idx3, comm, perp = pl.pallas_call(
        _vq_body,
        grid=grid,
        in_specs=[
            pl.BlockSpec((_B_BLK, d, _TOK_BLK), lambda i, j: (i, 0, j)),
            pl.BlockSpec((_K, d), lambda i, j: (0, 0)),
            pl.BlockSpec((_K, d), lambda i, j: (0, 0)),
            pl.BlockSpec((_K, d), lambda i, j: (0, 0)),
        ],
        out_specs=[
            pl.BlockSpec((_B_BLK, d, _TOK_BLK), lambda i, j: (i, 0, j)),
            pl.BlockSpec((_B_BLK, 1, _TOK_BLK), lambda i, j: (i, 0, j)),
            pl.BlockSpec((1, 1), lambda i, j: (0, 0)),
            pl.BlockSpec((1, 1), lambda i, j: (0, 0)),
        ],
        out_shape=[
            jax.ShapeDtypeStruct((b, d, t), jnp.float32),
            jax.ShapeDtypeStruct((b, 1, t), jnp.int32),
            jax.ShapeDtypeStruct((1, 1), jnp.float32),
            jax.ShapeDtypeStruct((1, 1), jnp.float32),
        ],
        scratch_shapes=[
            pltpu.VMEM((_K, _TOK_BLK), jnp.int16),
            pltpu.SMEM((1,), jnp.float32),
        ],
        compiler_params=pltpu.CompilerParams(
            dimension_semantics=("arbitrary", "arbitrary"),
        ),
    )(x, codebook, codebook * 2.0, codebook.astype(jnp.bfloat16))

    indices_2d = idx3.reshape(b, t)
    codebook_loss = jnp.zeros((), dtype=jnp.float32)
    return (q, indices_2d, codebook_loss, comm.reshape(()), perp.reshape(()))
